# baseline (speedup 1.0000x reference)
import functools

import jax
import jax.numpy as jnp
from jax import lax
from jax.experimental import pallas as pl
from jax.experimental.pallas import tpu as pltpu
from jax.experimental.pallas import tpu_sc as plsc

BATCH = 16384
EMB_K = 16
NROWS = 1000000
NC = 2
NS = 16
NW = NC * NS
BPW = BATCH // NW
CHUNK = 128
NCHUNK = BPW // CHUNK
JPC = CHUNK // 16


def _mf_body(xf_hbm, wf_hbm, hf_hbm, out_hbm, emb_hbm,
             xv, uidx, vidx, ugidx, vgidx, urt, vrt, outv, gsem):
    wid = lax.axis_index("s") * NC + lax.axis_index("c")
    base = wid * BPW

    pltpu.sync_copy(xf_hbm.at[pl.ds(wid * 8, 8), :], xv)

    iota = lax.iota(jnp.int32, 16)
    iota2 = iota * 2

    for j in range(BPW // 16):
        t = j // JPC
        off = (j % JPC) * 16
        row = jnp.full((16,), j // 4, jnp.int32)
        col = iota2 + 32 * (j % 4)
        uidx[t, pl.ds(off, 16)] = plsc.load_gather(xv, [row, col])
        vidx[t, pl.ds(off, 16)] = plsc.load_gather(xv, [row, col + 1])

    for t in range(NCHUNK):
        for off in range(0, CHUNK, 16):
            u = uidx[t, pl.ds(off, 16)]
            v = vidx[t, pl.ds(off, 16)]
            for k in range(EMB_K):
                ugidx[t, k, pl.ds(off, 16)] = u + (k * NROWS)
                vgidx[t, k, pl.ds(off, 16)] = v + (k * NROWS)

    copies = []
    for t in range(NCHUNK):
        dst = pl.ds(t * CHUNK, CHUNK)
        for k in range(EMB_K):
            copies.append(pltpu.async_copy(
                wf_hbm.at[ugidx.at[t, k]], urt.at[k, dst], gsem))
            copies.append(pltpu.async_copy(
                hf_hbm.at[vgidx.at[t, k]], vrt.at[k, dst], gsem))
    for cp in copies:
        cp.wait()

    def s_body(s, carry):
        sl = pl.ds(pl.multiple_of(s * 16, 16), 16)
        acc = urt[0, sl] * vrt[0, sl]
        for k in range(1, EMB_K):
            acc = acc + urt[k, sl] * vrt[k, sl]
        outv[s] = acc
        return carry

    lax.fori_loop(0, BPW // 16, s_body, 0)

    pltpu.sync_copy(outv, out_hbm.at[pl.ds(wid * (BPW // 16), BPW // 16), :])

    pltpu.sync_copy(urt, emb_hbm.at[pl.ds(0, EMB_K), pl.ds(base, BPW)])
    pltpu.sync_copy(vrt, emb_hbm.at[pl.ds(EMB_K, EMB_K), pl.ds(base, BPW)])


@jax.jit
def _mf(x, W, H):
    mesh = plsc.VectorSubcoreMesh(core_axis_name="c", subcore_axis_name="s")
    f = functools.partial(
        pl.kernel,
        mesh=mesh,
        compiler_params=pltpu.CompilerParams(
            needs_layout_passes=False, use_tc_tiling_on_sc=False),
        out_type=(
            jax.ShapeDtypeStruct((BATCH // 16, 16), jnp.float32),
            jax.ShapeDtypeStruct((2 * EMB_K, BATCH), jnp.float32),
        ),
        scratch_types=[
            pltpu.VMEM((8, 128), jnp.int32),
            pltpu.VMEM((NCHUNK, CHUNK), jnp.int32),
            pltpu.VMEM((NCHUNK, CHUNK), jnp.int32),
            pltpu.VMEM((NCHUNK, EMB_K, CHUNK), jnp.int32),
            pltpu.VMEM((NCHUNK, EMB_K, CHUNK), jnp.int32),
            pltpu.VMEM((EMB_K, BPW), jnp.float32),
            pltpu.VMEM((EMB_K, BPW), jnp.float32),
            pltpu.VMEM((BPW // 16, 16), jnp.float32),
            pltpu.SemaphoreType.DMA,
        ],
    )(_mf_body)
    return f(x.reshape(2 * BATCH // 128, 128),
             W.T.reshape(EMB_K * NROWS), H.T.reshape(EMB_K * NROWS))


def kernel(x, W, H):
    out2, embt = _mf(x, W, H)
    return (out2.reshape(BATCH), embt.T)


# speedup vs baseline: 15.4417x; 15.4417x over previous
import functools

import jax
import jax.numpy as jnp
from jax import lax
from jax.experimental import pallas as pl
from jax.experimental.pallas import tpu as pltpu
from jax.experimental.pallas import tpu_sc as plsc

BATCH = 16384
EMB_K = 16
NROWS = 1000000
NROWSP = 1000064
CUT = 999936
TB = EMB_K * NROWSP
FLAT = TB + (NROWS - CUT) * EMB_K
NC = 2
NS = 16
NW = NC * NS
BPW = BATCH // NW
CHUNK = 128
NCHUNK = BPW // CHUNK
JPC = CHUNK // 16

NTILEF = CUT // 128
NRANGE = 8
RSPAN = (NTILEF + NRANGE - 1) // NRANGE
ZC = 64
NCHD = (RSPAN + ZC - 1) // ZC


def _detile_body(wt_hbm, ht_hbm, wtl_hbm, htl_hbm, wf_hbm, hf_hbm,
                 buf, tbuf, sem):
    wid = lax.axis_index("s") * NC + lax.axis_index("c")
    tbl = wid // 16
    slab = (wid // 8) % 2
    rng = wid % 8
    last = NTILEF - ZC

    def chunk(j, carry):
        c0 = jnp.minimum(rng * RSPAN + j * ZC, last)

        def do(t_hbm, f_hbm):
            pltpu.async_copy(
                t_hbm.at[pl.ds(slab * 8, 8), pl.ds(c0 * 128, ZC * 128)],
                buf, sem).wait()
            for r in range(8):
                pltpu.sync_copy(
                    buf.at[r, :],
                    f_hbm.at[pl.ds((slab * 8 + r) * NROWSP + c0 * 128,
                                   ZC * 128)])

        @pl.when(tbl == 0)
        def _():
            do(wt_hbm, wf_hbm)

        @pl.when(tbl == 1)
        def _():
            do(ht_hbm, hf_hbm)

        return carry

    lax.fori_loop(0, NCHD, chunk, 0)

    @pl.when(wid == 0)
    def _():
        pltpu.sync_copy(wtl_hbm, tbuf)
        for r in range(8):
            pltpu.sync_copy(tbuf.at[r, :], wf_hbm.at[pl.ds(TB + r * 128, 128)])

    @pl.when(wid == 1)
    def _():
        pltpu.sync_copy(htl_hbm, tbuf)
        for r in range(8):
            pltpu.sync_copy(tbuf.at[r, :], hf_hbm.at[pl.ds(TB + r * 128, 128)])


def _mf_body(xf_hbm, wf_hbm, hf_hbm, out_hbm, emb_hbm,
             xv, uidx, vidx, ugidx, vgidx, urt, vrt, outv, gsem):
    wid = lax.axis_index("s") * NC + lax.axis_index("c")
    base = wid * BPW

    pltpu.sync_copy(xf_hbm.at[pl.ds(wid * 8, 8), :], xv)

    iota = lax.iota(jnp.int32, 16)
    iota2 = iota * 2

    for j in range(BPW // 16):
        t = j // JPC
        off = (j % JPC) * 16
        row = jnp.full((16,), j // 4, jnp.int32)
        col = iota2 + 32 * (j % 4)
        uidx[t, pl.ds(off, 16)] = plsc.load_gather(xv, [row, col])
        vidx[t, pl.ds(off, 16)] = plsc.load_gather(xv, [row, col + 1])

    for t in range(NCHUNK):
        for off in range(0, CHUNK, 16):
            u = uidx[t, pl.ds(off, 16)]
            v = vidx[t, pl.ds(off, 16)]
            um = u < CUT
            vm = v < CUT
            u16 = u * 16 + (TB - CUT * 16)
            v16 = v * 16 + (TB - CUT * 16)
            for k in range(EMB_K):
                ugidx[t, k, pl.ds(off, 16)] = jnp.where(
                    um, u + (k * NROWSP), u16 + k)
                vgidx[t, k, pl.ds(off, 16)] = jnp.where(
                    vm, v + (k * NROWSP), v16 + k)

    copies = []
    for t in range(NCHUNK):
        dst = pl.ds(t * CHUNK, CHUNK)
        for k in range(EMB_K):
            copies.append(pltpu.async_copy(
                wf_hbm.at[ugidx.at[t, k]], urt.at[k, dst], gsem))
            copies.append(pltpu.async_copy(
                hf_hbm.at[vgidx.at[t, k]], vrt.at[k, dst], gsem))
    for cp in copies:
        cp.wait()

    def s_body(s, carry):
        sl = pl.ds(pl.multiple_of(s * 16, 16), 16)
        acc = urt[0, sl] * vrt[0, sl]
        for k in range(1, EMB_K):
            acc = acc + urt[k, sl] * vrt[k, sl]
        outv[s] = acc
        return carry

    lax.fori_loop(0, BPW // 16, s_body, 0)

    pltpu.sync_copy(outv, out_hbm.at[pl.ds(wid * (BPW // 16), BPW // 16), :])

    pltpu.sync_copy(urt, emb_hbm.at[pl.ds(0, EMB_K), pl.ds(base, BPW)])
    pltpu.sync_copy(vrt, emb_hbm.at[pl.ds(EMB_K, EMB_K), pl.ds(base, BPW)])


@jax.jit
def _mf(x, W, H):
    mesh = plsc.VectorSubcoreMesh(core_axis_name="c", subcore_axis_name="s")

    detile = functools.partial(
        pl.kernel,
        mesh=mesh,
        compiler_params=pltpu.CompilerParams(
            needs_layout_passes=False, use_tc_tiling_on_sc=True),
        out_type=(
            jax.ShapeDtypeStruct((FLAT,), jnp.float32),
            jax.ShapeDtypeStruct((FLAT,), jnp.float32),
        ),
        scratch_types=[
            pltpu.VMEM((8, ZC * 128), jnp.float32),
            pltpu.VMEM((8, 128), jnp.float32),
            pltpu.SemaphoreType.DMA,
        ],
    )(_detile_body)

    gather = functools.partial(
        pl.kernel,
        mesh=mesh,
        compiler_params=pltpu.CompilerParams(
            needs_layout_passes=False, use_tc_tiling_on_sc=False),
        out_type=(
            jax.ShapeDtypeStruct((BATCH // 16, 16), jnp.float32),
            jax.ShapeDtypeStruct((2 * EMB_K, BATCH), jnp.float32),
        ),
        scratch_types=[
            pltpu.VMEM((8, 128), jnp.int32),
            pltpu.VMEM((NCHUNK, CHUNK), jnp.int32),
            pltpu.VMEM((NCHUNK, CHUNK), jnp.int32),
            pltpu.VMEM((NCHUNK, EMB_K, CHUNK), jnp.int32),
            pltpu.VMEM((NCHUNK, EMB_K, CHUNK), jnp.int32),
            pltpu.VMEM((EMB_K, BPW), jnp.float32),
            pltpu.VMEM((EMB_K, BPW), jnp.float32),
            pltpu.VMEM((BPW // 16, 16), jnp.float32),
            pltpu.SemaphoreType.DMA,
        ],
    )(_mf_body)

    wtail = W[CUT:, :].reshape(8, 128)
    htail = H[CUT:, :].reshape(8, 128)
    wf, hf = detile(W.T, H.T, wtail, htail)
    return gather(x.reshape(2 * BATCH // 128, 128), wf, hf)


def kernel(x, W, H):
    out2, embt = _mf(x, W, H)
    return (out2.reshape(BATCH), embt.T)


# speedup vs baseline: 16.4663x; 1.0664x over previous
import functools

import jax
import jax.numpy as jnp
from jax import lax
from jax.experimental import pallas as pl
from jax.experimental.pallas import tpu as pltpu
from jax.experimental.pallas import tpu_sc as plsc

BATCH = 16384
EMB_K = 16
NROWS = 1000000
NROWSP = 1000064
CUT = 999936
TB = EMB_K * NROWSP
FLAT = TB + (NROWS - CUT) * EMB_K
NC = 2
NS = 16
NW = NC * NS
BPW = BATCH // NW
CHUNK = 128
NCHUNK = BPW // CHUNK
JPC = CHUNK // 16

NTILEF = CUT // 128
NRANGE = 8
RSPAN = (NTILEF + NRANGE - 1) // NRANGE
ZC = 60
NCHD = (RSPAN + ZC - 1) // ZC


def _detile_body(wt_hbm, ht_hbm, wtl_hbm, htl_hbm, wf_hbm, hf_hbm,
                 buf, tbuf, isem, osem):
    wid = lax.axis_index("s") * NC + lax.axis_index("c")
    tbl = wid // 16
    slab = (wid // 8) % 2
    rng = wid % 8
    last = NTILEF - ZC

    def c_at(j):
        return jnp.minimum(rng * RSPAN + j * ZC, last)

    def do(t_hbm, f_hbm):
        def fire_in(j, b):
            return pltpu.async_copy(
                t_hbm.at[pl.ds(slab * 8, 8), pl.ds(c_at(j) * 128, ZC * 128)],
                buf.at[b], isem)

        pend_in = {0: fire_in(0, 0)}
        pend_out = {0: [], 1: []}
        for j in range(NCHD):
            b = j % 2
            pend_in.pop(j).wait()
            if j + 1 < NCHD:
                for cp in pend_out[1 - b]:
                    cp.wait()
                pend_out[1 - b] = []
                pend_in[j + 1] = fire_in(j + 1, 1 - b)
            c0 = c_at(j)
            pend_out[b] = [
                pltpu.async_copy(
                    buf.at[b, r],
                    f_hbm.at[pl.ds((slab * 8 + r) * NROWSP + c0 * 128,
                                   ZC * 128)],
                    osem)
                for r in range(8)
            ]
        for cps in pend_out.values():
            for cp in cps:
                cp.wait()

    @pl.when(tbl == 0)
    def _():
        do(wt_hbm, wf_hbm)

    @pl.when(tbl == 1)
    def _():
        do(ht_hbm, hf_hbm)

    @pl.when(wid == 0)
    def _():
        pltpu.sync_copy(wtl_hbm, tbuf)
        for r in range(8):
            pltpu.sync_copy(tbuf.at[r, :], wf_hbm.at[pl.ds(TB + r * 128, 128)])

    @pl.when(wid == 1)
    def _():
        pltpu.sync_copy(htl_hbm, tbuf)
        for r in range(8):
            pltpu.sync_copy(tbuf.at[r, :], hf_hbm.at[pl.ds(TB + r * 128, 128)])


def _mf_body(xf_hbm, wf_hbm, hf_hbm, out_hbm, emb_hbm,
             xv, ugidx, vgidx, urt, vrt, outv, gsem):
    wid = lax.axis_index("s") * NC + lax.axis_index("c")
    base = wid * BPW

    pltpu.sync_copy(xf_hbm.at[pl.ds(wid * 8, 8), :], xv)

    for t in range(NCHUNK):
        for off in range(0, CHUNK, 16):
            u = xv[2 * t, pl.ds(off, 16)]
            v = xv[2 * t + 1, pl.ds(off, 16)]
            um = u < CUT
            vm = v < CUT
            u16 = u * 16 + (TB - CUT * 16)
            v16 = v * 16 + (TB - CUT * 16)
            for k in range(EMB_K):
                ugidx[t, k, pl.ds(off, 16)] = jnp.where(
                    um, u + (k * NROWSP), u16 + k)
                vgidx[t, k, pl.ds(off, 16)] = jnp.where(
                    vm, v + (k * NROWSP), v16 + k)

    copies = []
    for t in range(NCHUNK):
        dst = pl.ds(t * CHUNK, CHUNK)
        for k in range(EMB_K):
            copies.append(pltpu.async_copy(
                wf_hbm.at[ugidx.at[t, k]], urt.at[k, dst], gsem))
            copies.append(pltpu.async_copy(
                hf_hbm.at[vgidx.at[t, k]], vrt.at[k, dst], gsem))
    for cp in copies:
        cp.wait()

    def s_body(s, carry):
        sl = pl.ds(pl.multiple_of(s * 16, 16), 16)
        acc = urt[0, sl] * vrt[0, sl]
        for k in range(1, EMB_K):
            acc = acc + urt[k, sl] * vrt[k, sl]
        outv[s] = acc
        return carry

    lax.fori_loop(0, BPW // 16, s_body, 0)

    pltpu.sync_copy(outv, out_hbm.at[pl.ds(wid * (BPW // 16), BPW // 16), :])

    pltpu.sync_copy(urt, emb_hbm.at[pl.ds(0, EMB_K), pl.ds(base, BPW)])
    pltpu.sync_copy(vrt, emb_hbm.at[pl.ds(EMB_K, EMB_K), pl.ds(base, BPW)])


@jax.jit
def _mf(x, W, H):
    mesh = plsc.VectorSubcoreMesh(core_axis_name="c", subcore_axis_name="s")

    detile = functools.partial(
        pl.kernel,
        mesh=mesh,
        compiler_params=pltpu.CompilerParams(
            needs_layout_passes=False, use_tc_tiling_on_sc=True),
        out_type=(
            jax.ShapeDtypeStruct((FLAT,), jnp.float32),
            jax.ShapeDtypeStruct((FLAT,), jnp.float32),
        ),
        scratch_types=[
            pltpu.VMEM((2, 8, ZC * 128), jnp.float32),
            pltpu.VMEM((8, 128), jnp.float32),
            pltpu.SemaphoreType.DMA,
            pltpu.SemaphoreType.DMA,
        ],
    )(_detile_body)

    gather = functools.partial(
        pl.kernel,
        mesh=mesh,
        compiler_params=pltpu.CompilerParams(
            needs_layout_passes=False, use_tc_tiling_on_sc=False),
        out_type=(
            jax.ShapeDtypeStruct((BATCH // 16, 16), jnp.float32),
            jax.ShapeDtypeStruct((2 * EMB_K, BATCH), jnp.float32),
        ),
        scratch_types=[
            pltpu.VMEM((8, 128), jnp.int32),
            pltpu.VMEM((NCHUNK, EMB_K, CHUNK), jnp.int32),
            pltpu.VMEM((NCHUNK, EMB_K, CHUNK), jnp.int32),
            pltpu.VMEM((EMB_K, BPW), jnp.float32),
            pltpu.VMEM((EMB_K, BPW), jnp.float32),
            pltpu.VMEM((BPW // 16, 16), jnp.float32),
            pltpu.SemaphoreType.DMA,
        ],
    )(_mf_body)

    wtail = W[CUT:, :].reshape(8, 128)
    htail = H[CUT:, :].reshape(8, 128)
    wf, hf = detile(W.T, H.T, wtail, htail)
    xn = jnp.transpose(x.T.reshape(2, BATCH // 128, 128), (1, 0, 2))
    return gather(xn.reshape(2 * BATCH // 128, 128), wf, hf)


def kernel(x, W, H):
    out2, embt = _mf(x, W, H)
    return (out2.reshape(BATCH), embt.T)


# speedup vs baseline: 18.0868x; 1.0984x over previous
import functools

import jax
import jax.numpy as jnp
from jax import lax
from jax.experimental import pallas as pl
from jax.experimental.pallas import tpu as pltpu
from jax.experimental.pallas import tpu_sc as plsc

BATCH = 16384
EMB_K = 16
NROWS = 1000000
NROWSP = 1000064
CUT = 999936
TB = EMB_K * NROWSP
FLAT = TB + (NROWS - CUT) * EMB_K
NC = 2
NS = 16
NW = NC * NS
BPW = BATCH // NW
CHUNK = 128
NCHUNK = BPW // CHUNK
JPC = CHUNK // 16

NTILEF = CUT // 128
NRANGE = 8
RSPAN = (NTILEF + NRANGE - 1) // NRANGE
ZC = 60
NCHD = (RSPAN + ZC - 1) // ZC


def _detile_body(wt_hbm, ht_hbm, wtl_hbm, htl_hbm, wf_hbm, hf_hbm,
                 buf, tbuf, isem, osem):
    wid = lax.axis_index("s") * NC + lax.axis_index("c")
    sid = lax.axis_index("s")
    tbl = wid // 16
    slab = (wid // 8) % 2
    rng = wid % 8
    last = NTILEF - ZC

    def c_at(j):
        return jnp.minimum(rng * RSPAN + j * ZC, last)

    def do(t_hbm, f_hbm):
        def fire_in(j, b):
            return pltpu.async_copy(
                t_hbm.at[pl.ds(slab * 8, 8), pl.ds(c_at(j) * 128, ZC * 128)],
                buf.at[sid, b], isem)

        pend_in = {0: fire_in(0, 0)}
        pend_out = {0: [], 1: []}
        for j in range(NCHD):
            b = j % 2
            pend_in.pop(j).wait()
            if j + 1 < NCHD:
                for cp in pend_out[1 - b]:
                    cp.wait()
                pend_out[1 - b] = []
                pend_in[j + 1] = fire_in(j + 1, 1 - b)
            c0 = c_at(j)
            pend_out[b] = [
                pltpu.async_copy(
                    buf.at[sid, b, r],
                    f_hbm.at[pl.ds((slab * 8 + r) * NROWSP + c0 * 128,
                                   ZC * 128)],
                    osem)
                for r in range(8)
            ]
        for cps in pend_out.values():
            for cp in cps:
                cp.wait()

    @pl.when(tbl == 0)
    def _():
        do(wt_hbm, wf_hbm)

    @pl.when(tbl == 1)
    def _():
        do(ht_hbm, hf_hbm)

    @pl.when(wid == 0)
    def _():
        pltpu.sync_copy(wtl_hbm, tbuf)
        for r in range(8):
            pltpu.sync_copy(tbuf.at[r, :], wf_hbm.at[pl.ds(TB + r * 128, 128)])

    @pl.when(wid == 1)
    def _():
        pltpu.sync_copy(htl_hbm, tbuf)
        for r in range(8):
            pltpu.sync_copy(tbuf.at[r, :], hf_hbm.at[pl.ds(TB + r * 128, 128)])


def _mf_body(xf_hbm, wf_hbm, hf_hbm, out_hbm, emb_hbm,
             xv, ugidx, vgidx, urt, vrt, outv, gsem):
    wid = lax.axis_index("s") * NC + lax.axis_index("c")
    base = wid * BPW

    pltpu.sync_copy(xf_hbm.at[pl.ds(wid * 8, 8), :], xv)

    for t in range(NCHUNK):
        for off in range(0, CHUNK, 16):
            u = xv[2 * t, pl.ds(off, 16)]
            v = xv[2 * t + 1, pl.ds(off, 16)]
            um = u < CUT
            vm = v < CUT
            u16 = u * 16 + (TB - CUT * 16)
            v16 = v * 16 + (TB - CUT * 16)
            for k in range(EMB_K):
                ugidx[t, k, pl.ds(off, 16)] = jnp.where(
                    um, u + (k * NROWSP), u16 + k)
                vgidx[t, k, pl.ds(off, 16)] = jnp.where(
                    vm, v + (k * NROWSP), v16 + k)

    copies = []
    for t in range(NCHUNK):
        dst = pl.ds(t * CHUNK, CHUNK)
        for k in range(EMB_K):
            copies.append(pltpu.async_copy(
                wf_hbm.at[ugidx.at[t, k]], urt.at[k, dst], gsem))
            copies.append(pltpu.async_copy(
                hf_hbm.at[vgidx.at[t, k]], vrt.at[k, dst], gsem))
    for cp in copies:
        cp.wait()

    def s_body(s, carry):
        sl = pl.ds(pl.multiple_of(s * 16, 16), 16)
        acc = urt[0, sl] * vrt[0, sl]
        for k in range(1, EMB_K):
            acc = acc + urt[k, sl] * vrt[k, sl]
        outv[s] = acc
        return carry

    lax.fori_loop(0, BPW // 16, s_body, 0)

    pltpu.sync_copy(outv, out_hbm.at[pl.ds(wid * (BPW // 16), BPW // 16), :])

    pltpu.sync_copy(urt, emb_hbm.at[pl.ds(0, EMB_K), pl.ds(base, BPW)])
    pltpu.sync_copy(vrt, emb_hbm.at[pl.ds(EMB_K, EMB_K), pl.ds(base, BPW)])


@jax.jit
def _mf(x, W, H):
    mesh = plsc.VectorSubcoreMesh(core_axis_name="c", subcore_axis_name="s")

    detile = functools.partial(
        pl.kernel,
        mesh=mesh,
        compiler_params=pltpu.CompilerParams(
            needs_layout_passes=False, use_tc_tiling_on_sc=True),
        out_type=(
            jax.ShapeDtypeStruct((FLAT,), jnp.float32),
            jax.ShapeDtypeStruct((FLAT,), jnp.float32),
        ),
        scratch_types=[
            pltpu.VMEM_SHARED((NS, 2, 8, ZC * 128), jnp.float32),
            pltpu.VMEM((8, 128), jnp.float32),
            pltpu.SemaphoreType.DMA,
            pltpu.SemaphoreType.DMA,
        ],
    )(_detile_body)

    gather = functools.partial(
        pl.kernel,
        mesh=mesh,
        compiler_params=pltpu.CompilerParams(
            needs_layout_passes=False, use_tc_tiling_on_sc=False),
        out_type=(
            jax.ShapeDtypeStruct((BATCH // 16, 16), jnp.float32),
            jax.ShapeDtypeStruct((2 * EMB_K, BATCH), jnp.float32),
        ),
        scratch_types=[
            pltpu.VMEM((8, 128), jnp.int32),
            pltpu.VMEM((NCHUNK, EMB_K, CHUNK), jnp.int32),
            pltpu.VMEM((NCHUNK, EMB_K, CHUNK), jnp.int32),
            pltpu.VMEM((EMB_K, BPW), jnp.float32),
            pltpu.VMEM((EMB_K, BPW), jnp.float32),
            pltpu.VMEM((BPW // 16, 16), jnp.float32),
            pltpu.SemaphoreType.DMA,
        ],
    )(_mf_body)

    wtail = W[CUT:, :].reshape(8, 128)
    htail = H[CUT:, :].reshape(8, 128)
    wf, hf = detile(W.T, H.T, wtail, htail)
    xn = jnp.transpose(x.T.reshape(2, BATCH // 128, 128), (1, 0, 2))
    return gather(xn.reshape(2 * BATCH // 128, 128), wf, hf)


def kernel(x, W, H):
    out2, embt = _mf(x, W, H)
    return (out2.reshape(BATCH), embt.T)


# speedup vs baseline: 18.1152x; 1.0016x over previous
import functools

import jax
import jax.numpy as jnp
from jax import lax
from jax.experimental import pallas as pl
from jax.experimental.pallas import tpu as pltpu
from jax.experimental.pallas import tpu_sc as plsc

BATCH = 16384
EMB_K = 16
NROWS = 1000000
NROWSP = 1000064
CUT = 999936
TB = EMB_K * NROWSP
FLAT = TB + (NROWS - CUT) * EMB_K
NC = 2
NS = 16
NW = NC * NS
BPW = BATCH // NW
CHUNK = 128
NCHUNK = BPW // CHUNK

NTILEF = CUT // 128
NRANGE = 8
RSPAN = (NTILEF + NRANGE - 1) // NRANGE
ZC = 60
NCHD = (RSPAN + ZC - 1) // ZC


def _detile_body(wt_hbm, ht_hbm, wtl_hbm, htl_hbm, wf_hbm, hf_hbm,
                 buf, tbuf, isem, osem):
    wid = lax.axis_index("s") * NC + lax.axis_index("c")
    sid = lax.axis_index("s")
    tbl = wid // 16
    slab = (wid // 8) % 2
    rng = wid % 8
    last = NTILEF - ZC

    def c_at(j):
        return jnp.minimum(rng * RSPAN + j * ZC, last)

    def do(t_hbm, f_hbm):
        def fire_in(j, b):
            return pltpu.async_copy(
                t_hbm.at[pl.ds(slab * 8, 8), pl.ds(c_at(j) * 128, ZC * 128)],
                buf.at[sid, b], isem)

        pend_in = {0: fire_in(0, 0)}
        pend_out = {0: [], 1: []}
        for j in range(NCHD):
            b = j % 2
            pend_in.pop(j).wait()
            if j + 1 < NCHD:
                for cp in pend_out[1 - b]:
                    cp.wait()
                pend_out[1 - b] = []
                pend_in[j + 1] = fire_in(j + 1, 1 - b)
            c0 = c_at(j)
            pend_out[b] = [
                pltpu.async_copy(
                    buf.at[sid, b, r],
                    f_hbm.at[pl.ds((slab * 8 + r) * NROWSP + c0 * 128,
                                   ZC * 128)],
                    osem)
                for r in range(8)
            ]
        for cps in pend_out.values():
            for cp in cps:
                cp.wait()

    @pl.when(tbl == 0)
    def _():
        do(wt_hbm, wf_hbm)

    @pl.when(tbl == 1)
    def _():
        do(ht_hbm, hf_hbm)

    @pl.when(wid == 0)
    def _():
        pltpu.sync_copy(wtl_hbm, tbuf)
        for r in range(8):
            pltpu.sync_copy(tbuf.at[r, :], wf_hbm.at[pl.ds(TB + r * 128, 128)])

    @pl.when(wid == 1)
    def _():
        pltpu.sync_copy(htl_hbm, tbuf)
        for r in range(8):
            pltpu.sync_copy(tbuf.at[r, :], hf_hbm.at[pl.ds(TB + r * 128, 128)])


def _mf_body(xf_hbm, wf_hbm, hf_hbm, out_hbm, emb_hbm,
             xv, ugidx, vgidx, urt, vrt, outv, gsem):
    wid = lax.axis_index("s") * NC + lax.axis_index("c")
    base = wid * BPW

    pltpu.sync_copy(xf_hbm.at[pl.ds(wid * 8, 8), :], xv)

    for t in range(NCHUNK):
        for off in range(0, CHUNK, 16):
            u = xv[2 * t, pl.ds(off, 16)]
            v = xv[2 * t + 1, pl.ds(off, 16)]
            um = u < CUT
            vm = v < CUT
            u16 = u * 16 + (TB - CUT * 16)
            v16 = v * 16 + (TB - CUT * 16)
            for k in range(EMB_K):
                ugidx[t, k, pl.ds(off, 16)] = jnp.where(
                    um, u + (k * NROWSP), u16 + k)
                vgidx[t, k, pl.ds(off, 16)] = jnp.where(
                    vm, v + (k * NROWSP), v16 + k)

    copies = []
    for t in range(NCHUNK):
        dst = pl.ds(t * CHUNK, CHUNK)
        for k in range(EMB_K):
            copies.append(pltpu.async_copy(
                wf_hbm.at[ugidx.at[t, k]], urt.at[k, dst], gsem))
            copies.append(pltpu.async_copy(
                hf_hbm.at[vgidx.at[t, k]], vrt.at[k, dst], gsem))
    for cp in copies:
        cp.wait()

    def s_body(s, carry):
        sl = pl.ds(pl.multiple_of(s * 16, 16), 16)
        acc = urt[0, sl] * vrt[0, sl]
        for k in range(1, EMB_K):
            acc = acc + urt[k, sl] * vrt[k, sl]
        outv[s] = acc
        return carry

    lax.fori_loop(0, BPW // 16, s_body, 0)

    pltpu.sync_copy(outv, out_hbm.at[pl.ds(wid * (BPW // 16), BPW // 16), :])

    pltpu.sync_copy(urt, emb_hbm.at[pl.ds(0, EMB_K), pl.ds(base, BPW)])
    pltpu.sync_copy(vrt, emb_hbm.at[pl.ds(EMB_K, EMB_K), pl.ds(base, BPW)])


@jax.jit
def _mf(x, W, H):
    mesh = plsc.VectorSubcoreMesh(core_axis_name="c", subcore_axis_name="s")

    detile = functools.partial(
        pl.kernel,
        mesh=mesh,
        compiler_params=pltpu.CompilerParams(
            needs_layout_passes=False, use_tc_tiling_on_sc=True),
        out_type=(
            jax.ShapeDtypeStruct((FLAT,), jnp.float32),
            jax.ShapeDtypeStruct((FLAT,), jnp.float32),
        ),
        scratch_types=[
            pltpu.VMEM_SHARED((NS, 2, 8, ZC * 128), jnp.float32),
            pltpu.VMEM((8, 128), jnp.float32),
            pltpu.SemaphoreType.DMA,
            pltpu.SemaphoreType.DMA,
        ],
    )(_detile_body)

    gather = functools.partial(
        pl.kernel,
        mesh=mesh,
        compiler_params=pltpu.CompilerParams(
            needs_layout_passes=False, use_tc_tiling_on_sc=False),
        out_type=(
            jax.ShapeDtypeStruct((BATCH // 16, 16), jnp.float32),
            jax.ShapeDtypeStruct((2 * EMB_K, BATCH), jnp.float32),
        ),
        scratch_types=[
            pltpu.VMEM((8, 128), jnp.int32),
            pltpu.VMEM((NCHUNK, EMB_K, CHUNK), jnp.int32),
            pltpu.VMEM((NCHUNK, EMB_K, CHUNK), jnp.int32),
            pltpu.VMEM((EMB_K, BPW), jnp.float32),
            pltpu.VMEM((EMB_K, BPW), jnp.float32),
            pltpu.VMEM((BPW // 16, 16), jnp.float32),
            pltpu.SemaphoreType.DMA,
        ],
    )(_mf_body)

    wtail = W[CUT:, :].reshape(8, 128)
    htail = H[CUT:, :].reshape(8, 128)
    wf, hf = detile(W.T, H.T, wtail, htail)
    xn = jnp.transpose(x.T.reshape(2, BATCH // 128, 128), (1, 0, 2))
    return gather(xn.reshape(2 * BATCH // 128, 128), wf, hf)


def kernel(x, W, H):
    out2, embt = _mf(x, W, H)
    return (out2.reshape(BATCH), embt.T)
